# baseline (speedup 1.0000x reference)
import functools

import jax
import jax.numpy as jnp
import numpy as np
from jax import lax
from jax.experimental import pallas as pl
from jax.experimental.pallas import tpu as pltpu
from jax.experimental.pallas import tpu_sc as plsc

N_NODES = 10000
NFEAT = 128
NWORDS = NFEAT // 2
N_CORES = 2
N_SUBCORES = 16
N_WORKERS = N_CORES * N_SUBCORES
LANES = 16
ROWS_PER_TILE = 624
TAIL_ROW0 = N_SUBCORES * ROWS_PER_TILE
TAIL_ROWS = N_NODES - TAIL_ROW0

CHUNK = 96
E_GROUPS = CHUNK // LANES
WINDOWS = NFEAT // (2 * LANES)
N_CHUNKS = 112
S_PHASE = 16
N_PHASES = N_CHUNKS // S_PHASE
E_PER_WORKER = N_CHUNKS * CHUNK

_PERM = np.empty(NFEAT, np.int32)
for _g in range(WINDOWS):
    for _i in range(LANES):
        _PERM[32 * _g + 2 * _i] = 32 * _g + _i
        _PERM[32 * _g + 2 * _i + 1] = 32 * _g + LANES + _i


def _pack_table(h_f32):
    hb = h_f32[:, _PERM].astype(jnp.bfloat16)
    return jax.lax.bitcast_convert_type(
        hb.reshape(N_NODES, NWORDS, 2), jnp.int32)


def _pack_edges(src, dst, w):
    n_real = src.shape[0] // N_WORKERS

    def shape(a):
        a = a.reshape(N_WORKERS, n_real)
        a = jnp.pad(a, ((0, 0), (0, E_PER_WORKER - n_real)))
        return a.reshape(N_WORKERS, N_CHUNKS, CHUNK)

    return jnp.stack([shape(src), shape(dst)], axis=2), shape(w)


def _sc_aggregate(h_packed, packed, packed_w):
    mesh = plsc.VectorSubcoreMesh(core_axis_name="c", subcore_axis_name="s")

    @functools.partial(
        pl.kernel,
        mesh=mesh,
        out_type=jax.ShapeDtypeStruct((N_CORES, N_NODES, NFEAT), jnp.float32),
        scratch_types=[
            pltpu.VMEM_SHARED((N_NODES, NFEAT), jnp.float32),
            pltpu.VMEM((S_PHASE, 2, CHUNK), jnp.int32),
            pltpu.VMEM((S_PHASE, CHUNK), jnp.float32),
            pltpu.VMEM((CHUNK, NWORDS), jnp.int32),
            pltpu.VMEM((CHUNK, NWORDS), jnp.int32),
            pltpu.VMEM((CHUNK, NWORDS), jnp.int32),
            pltpu.VMEM((CHUNK, NFEAT), jnp.float32),
            pltpu.VMEM((CHUNK, NFEAT), jnp.float32),
            pltpu.SemaphoreType.DMA,
            pltpu.SemaphoreType.DMA,
            pltpu.SemaphoreType.DMA,
            pltpu.SemaphoreType.DMA,
            pltpu.SemaphoreType.DMA,
        ],
        compiler_params=pltpu.CompilerParams(use_tc_tiling_on_sc=False),
    )
    def agg(h_hbm, packed_hbm, packedw_hbm, out_hbm,
            acc, stage, stage_w, mb0, mb1, mb2, fb0, fb1,
            g0, g1, g2, s0, s1):
        c = lax.axis_index("c")
        s = lax.axis_index("s")
        wid = c * N_SUBCORES + s
        mbufs = (mb0, mb1, mb2)
        fbufs = (fb0, fb1)
        gsem = (g0, g1, g2)
        ssem = (s0, s1)
        zero16 = jnp.zeros((LANES,), jnp.float32)
        himask = jnp.full((LANES,), -65536, jnp.int32)

        def zero_row(r, carry):
            for f in range(NFEAT // LANES):
                fb0[r, pl.ds(f * LANES, LANES)] = zero16
            return carry

        lax.fori_loop(0, CHUNK, zero_row, 0)
        row0 = s * ROWS_PER_TILE
        for jz in range(ROWS_PER_TILE // CHUNK):
            pltpu.sync_copy(fb0, acc.at[pl.ds(row0 + jz * CHUNK, CHUNK)])
        rem0 = (ROWS_PER_TILE // CHUNK) * CHUNK
        rem = ROWS_PER_TILE - rem0
        if rem:
            pltpu.sync_copy(fb0.at[pl.ds(0, rem)],
                            acc.at[pl.ds(row0 + rem0, rem)])

        @pl.when(s == N_SUBCORES - 1)
        def _zero_tail():
            pltpu.sync_copy(fb0.at[pl.ds(0, TAIL_ROWS)],
                            acc.at[pl.ds(TAIL_ROW0, TAIL_ROWS)])

        plsc.subcore_barrier()

        def issue_gather(j, b):
            pltpu.async_copy(h_hbm.at[stage.at[j, 0]], mbufs[b], gsem[b])

        def wait_gather(j, b):
            pltpu.make_async_copy(h_hbm.at[stage.at[j, 0]],
                                  mbufs[b], gsem[b]).wait()

        def issue_scatter(j, b):
            pltpu.async_copy(fbufs[b], acc.at[stage.at[j, 1]], ssem[b],
                             add=True)

        def wait_scatter(j, b):
            pltpu.make_async_copy(fbufs[b], acc.at[stage.at[j, 1]],
                                  ssem[b]).wait()

        def scale(j, gb, fbb):
            mb = mbufs[gb]
            fb = fbufs[fbb]

            def scale_g(g, carry):
                wv = stage_w[j, pl.ds(g * LANES, LANES)]
                for r2 in range(LANES):
                    e = g * LANES + r2
                    wr = wv[r2]
                    for q in range(WINDOWS):
                        word = mb[e, pl.ds(LANES * q, LANES)]
                        lo_b = word << 16
                        hi_b = word & himask
                        lo = jax.lax.bitcast_convert_type(lo_b, jnp.float32)
                        hi = jax.lax.bitcast_convert_type(hi_b, jnp.float32)
                        fb[e, pl.ds(32 * q, LANES)] = lo * wr
                        fb[e, pl.ds(32 * q + LANES, LANES)] = hi * wr
                return carry

            lax.fori_loop(0, E_GROUPS, scale_g, 0)

        def phase_body(ph, carry):
            pltpu.sync_copy(packed_hbm.at[wid, pl.ds(ph * S_PHASE, S_PHASE)],
                            stage)
            pltpu.sync_copy(packedw_hbm.at[wid, pl.ds(ph * S_PHASE, S_PHASE)],
                            stage_w)
            issue_gather(0, 0)
            issue_gather(1, 1)
            wait_gather(0, 0)
            issue_gather(2, 2)
            scale(0, 0, 0)
            issue_scatter(0, 0)
            wait_gather(1, 1)
            issue_gather(3, 0)
            scale(1, 1, 1)
            issue_scatter(1, 1)

            def six(t, cc):
                for r in range(6):
                    j = 2 + 6 * t + r
                    gb = (2 + r) % 3
                    fbb = r % 2
                    wait_scatter(j - 2, fbb)
                    wait_gather(j, gb)
                    issue_gather(j + 2, (4 + r) % 3)
                    scale(j, gb, fbb)
                    issue_scatter(j, fbb)
                return cc

            lax.fori_loop(0, (S_PHASE - 4) // 6, six, 0)

            jj = S_PHASE - 2
            wait_scatter(jj - 2, jj % 2)
            wait_gather(jj, jj % 3)
            scale(jj, jj % 3, jj % 2)
            issue_scatter(jj, jj % 2)
            jj = S_PHASE - 1
            wait_scatter(jj - 2, jj % 2)
            wait_gather(jj, jj % 3)
            scale(jj, jj % 3, jj % 2)
            issue_scatter(jj, jj % 2)
            wait_scatter(S_PHASE - 2, (S_PHASE - 2) % 2)
            wait_scatter(S_PHASE - 1, (S_PHASE - 1) % 2)
            return carry

        lax.fori_loop(0, N_PHASES, phase_body, 0)

        plsc.subcore_barrier()
        pltpu.sync_copy(acc.at[pl.ds(row0, ROWS_PER_TILE)],
                        out_hbm.at[c, pl.ds(row0, ROWS_PER_TILE)])

        @pl.when(s == N_SUBCORES - 1)
        def _copy_tail():
            pltpu.sync_copy(acc.at[pl.ds(TAIL_ROW0, TAIL_ROWS)],
                            out_hbm.at[c, pl.ds(TAIL_ROW0, TAIL_ROWS)])

    return agg(h_packed, packed, packed_w)


def _combine_matmul(parts, W, relu):
    rows_blk = 2000

    def mm(p_ref, w_ref, o_ref):
        a = p_ref[0] + p_ref[1]
        y = jnp.dot(a, w_ref[...], preferred_element_type=jnp.float32)
        if relu:
            y = jnp.maximum(y, 0.0)
        o_ref[...] = y

    return pl.pallas_call(
        mm,
        grid=(N_NODES // rows_blk,),
        in_specs=[
            pl.BlockSpec((N_CORES, rows_blk, NFEAT), lambda i: (0, i, 0)),
            pl.BlockSpec((NFEAT, NFEAT), lambda i: (0, 0)),
        ],
        out_specs=pl.BlockSpec((rows_blk, NFEAT), lambda i: (i, 0)),
        out_shape=jax.ShapeDtypeStruct((N_NODES, NFEAT), jnp.float32),
    )(parts, W)


def kernel(x, edge_index1, edge_index2, edge_weight1, edge_weight2, W1, W2):
    packed1, pw1 = _pack_edges(edge_index1[0].astype(jnp.int32),
                               edge_index1[1].astype(jnp.int32), edge_weight1)
    packed2, pw2 = _pack_edges(edge_index2[0].astype(jnp.int32),
                               edge_index2[1].astype(jnp.int32), edge_weight2)

    p1 = _sc_aggregate(_pack_table(x), packed1, pw1)
    h = _combine_matmul(p1, W1, relu=True)
    p2 = _sc_aggregate(_pack_table(h), packed2, pw2)
    return _combine_matmul(p2, W2, relu=False)


# speedup vs baseline: 3.0873x; 3.0873x over previous
import functools

import jax
import jax.numpy as jnp
from jax import lax
from jax.experimental import pallas as pl
from jax.experimental.pallas import tpu as pltpu
from jax.experimental.pallas import tpu_sc as plsc

N_NODES = 10000
NFEAT = 128
N_CORES = 2
N_SUBCORES = 16
N_WORKERS = N_CORES * N_SUBCORES
LANES = 16
F_CHUNKS = NFEAT // LANES
ROWS_PER_TILE = 624
TAIL_ROW0 = N_SUBCORES * ROWS_PER_TILE
TAIL_ROWS = N_NODES - TAIL_ROW0

CHUNK = 96
G_CHUNKS = CHUNK // LANES
N_CHUNKS = 105
S_PHASE = 35
N_PHASES = N_CHUNKS // S_PHASE
E_PER_WORKER = N_CHUNKS * CHUNK


def _pack_edges(src, dst, w):
    n_real = src.shape[0] // N_WORKERS

    def shape(a):
        a = a.reshape(N_WORKERS, n_real)
        a = jnp.pad(a, ((0, 0), (0, E_PER_WORKER - n_real)))
        return a.reshape(N_WORKERS, N_CHUNKS, CHUNK)

    return jnp.stack([shape(src), shape(dst)], axis=2), shape(w)


def _sc_aggregate(h, packed, packed_w):
    mesh = plsc.VectorSubcoreMesh(core_axis_name="c", subcore_axis_name="s")

    @functools.partial(
        pl.kernel,
        mesh=mesh,
        out_type=jax.ShapeDtypeStruct((N_CORES, N_NODES, NFEAT), jnp.float32),
        scratch_types=[
            pltpu.VMEM_SHARED((N_NODES, NFEAT), jnp.float32),
            pltpu.VMEM((S_PHASE, 2, CHUNK), jnp.int32),
            pltpu.VMEM((S_PHASE, CHUNK), jnp.float32),
            pltpu.VMEM((CHUNK, NFEAT), jnp.float32),
            pltpu.VMEM((CHUNK, NFEAT), jnp.float32),
            pltpu.VMEM((CHUNK, NFEAT), jnp.float32),
            pltpu.SemaphoreType.DMA,
            pltpu.SemaphoreType.DMA,
            pltpu.SemaphoreType.DMA,
            pltpu.SemaphoreType.DMA,
            pltpu.SemaphoreType.DMA,
            pltpu.SemaphoreType.DMA,
        ],
        compiler_params=pltpu.CompilerParams(use_tc_tiling_on_sc=False),
    )
    def agg(h_hbm, packed_hbm, packedw_hbm, out_hbm,
            acc, stage, stage_w, m0, m1, m2, g0, g1, g2, s0, s1, s2):
        c = lax.axis_index("c")
        s = lax.axis_index("s")
        wid = c * N_SUBCORES + s
        msgs = (m0, m1, m2)
        gsem = (g0, g1, g2)
        ssem = (s0, s1, s2)
        zero16 = jnp.zeros((LANES,), jnp.float32)

        def zero_row(r, carry):
            for f in range(F_CHUNKS):
                m0[r, pl.ds(f * LANES, LANES)] = zero16
            return carry

        lax.fori_loop(0, CHUNK, zero_row, 0)
        row0 = s * ROWS_PER_TILE
        for jz in range(ROWS_PER_TILE // CHUNK):
            pltpu.sync_copy(m0, acc.at[pl.ds(row0 + jz * CHUNK, CHUNK)])
        rem0 = (ROWS_PER_TILE // CHUNK) * CHUNK
        rem = ROWS_PER_TILE - rem0
        if rem:
            pltpu.sync_copy(m0.at[pl.ds(0, rem)],
                            acc.at[pl.ds(row0 + rem0, rem)])

        @pl.when(s == N_SUBCORES - 1)
        def _zero_tail():
            pltpu.sync_copy(m0.at[pl.ds(0, TAIL_ROWS)],
                            acc.at[pl.ds(TAIL_ROW0, TAIL_ROWS)])

        plsc.subcore_barrier()

        def issue_gather(j, b):
            pltpu.async_copy(h_hbm.at[stage.at[j, 0]], msgs[b], gsem[b])

        def wait_gather(j, b):
            pltpu.make_async_copy(h_hbm.at[stage.at[j, 0]],
                                  msgs[b], gsem[b]).wait()

        def issue_scatter(j, b):
            pltpu.async_copy(msgs[b], acc.at[stage.at[j, 1]], ssem[b],
                             add=True)

        def wait_scatter(j, b):
            pltpu.make_async_copy(msgs[b], acc.at[stage.at[j, 1]],
                                  ssem[b]).wait()

        def scale(j, b):
            mb = msgs[b]

            def scale_g(g, carry):
                wv = stage_w[j, pl.ds(g * LANES, LANES)]
                for r2 in range(LANES):
                    e = g * LANES + r2
                    wr = wv[r2]
                    for f in range(F_CHUNKS):
                        sl = pl.ds(f * LANES, LANES)
                        mb[e, sl] = mb[e, sl] * wr
                return carry

            lax.fori_loop(0, G_CHUNKS, scale_g, 0)

        def process(j, b):
            wait_gather(j, b)
            scale(j, b)
            issue_scatter(j, b)

        def phase_body(ph, carry):
            pltpu.sync_copy(packed_hbm.at[wid, pl.ds(ph * S_PHASE, S_PHASE)],
                            stage)
            pltpu.sync_copy(packedw_hbm.at[wid, pl.ds(ph * S_PHASE, S_PHASE)],
                            stage_w)
            issue_gather(0, 0)
            issue_gather(1, 1)
            process(0, 0)
            issue_gather(2, 2)
            process(1, 1)
            wait_scatter(0, 0)
            issue_gather(3, 0)
            process(2, 2)
            wait_scatter(1, 1)
            issue_gather(4, 1)

            def triple(t, cc):
                for r in range(3):
                    j = 3 + 3 * t + r
                    b = r
                    process(j, b)
                    nb = (r + 2) % 3
                    wait_scatter(j - 1, nb)
                    issue_gather(j + 2, nb)
                return cc

            lax.fori_loop(0, (S_PHASE - 5) // 3, triple, 0)

            process(S_PHASE - 2, 0)
            process(S_PHASE - 1, 1)
            wait_scatter(S_PHASE - 3, 2)
            wait_scatter(S_PHASE - 2, 0)
            wait_scatter(S_PHASE - 1, 1)
            return carry

        lax.fori_loop(0, N_PHASES, phase_body, 0)

        plsc.subcore_barrier()
        pltpu.sync_copy(acc.at[pl.ds(row0, ROWS_PER_TILE)],
                        out_hbm.at[c, pl.ds(row0, ROWS_PER_TILE)])

        @pl.when(s == N_SUBCORES - 1)
        def _copy_tail():
            pltpu.sync_copy(acc.at[pl.ds(TAIL_ROW0, TAIL_ROWS)],
                            out_hbm.at[c, pl.ds(TAIL_ROW0, TAIL_ROWS)])

    return agg(h, packed, packed_w)


def _combine_matmul(parts, W, relu):
    rows_blk = 1000

    def mm(p_ref, w_ref, o_ref):
        a = p_ref[0] + p_ref[1]
        y = jnp.dot(a, w_ref[...], preferred_element_type=jnp.float32)
        if relu:
            y = jnp.maximum(y, 0.0)
        o_ref[...] = y

    return pl.pallas_call(
        mm,
        grid=(N_NODES // rows_blk,),
        in_specs=[
            pl.BlockSpec((N_CORES, rows_blk, NFEAT), lambda i: (0, i, 0)),
            pl.BlockSpec((NFEAT, NFEAT), lambda i: (0, 0)),
        ],
        out_specs=pl.BlockSpec((rows_blk, NFEAT), lambda i: (i, 0)),
        out_shape=jax.ShapeDtypeStruct((N_NODES, NFEAT), jnp.float32),
    )(parts, W)


def kernel(x, edge_index1, edge_index2, edge_weight1, edge_weight2, W1, W2):
    packed1, pw1 = _pack_edges(edge_index1[0].astype(jnp.int32),
                               edge_index1[1].astype(jnp.int32), edge_weight1)
    packed2, pw2 = _pack_edges(edge_index2[0].astype(jnp.int32),
                               edge_index2[1].astype(jnp.int32), edge_weight2)

    p1 = _sc_aggregate(x, packed1, pw1)
    h = _combine_matmul(p1, W1, relu=True)
    p2 = _sc_aggregate(h, packed2, pw2)
    return _combine_matmul(p2, W2, relu=False)


# speedup vs baseline: 3.0886x; 1.0004x over previous
import functools

import jax
import jax.numpy as jnp
from jax import lax
from jax.experimental import pallas as pl
from jax.experimental.pallas import tpu as pltpu
from jax.experimental.pallas import tpu_sc as plsc

N_NODES = 10000
NFEAT = 128
N_CORES = 2
N_SUBCORES = 16
N_WORKERS = N_CORES * N_SUBCORES
LANES = 16
F_CHUNKS = NFEAT // LANES
ROWS_PER_TILE = 624
TAIL_ROW0 = N_SUBCORES * ROWS_PER_TILE
TAIL_ROWS = N_NODES - TAIL_ROW0

CHUNK = 96
G_CHUNKS = CHUNK // LANES
N_CHUNKS = 105
S_PHASE = 35
N_PHASES = N_CHUNKS // S_PHASE
E_PER_WORKER = N_CHUNKS * CHUNK


def _pack_edges(src, dst, w):
    n_real = src.shape[0] // N_WORKERS

    def shape(a):
        a = a.reshape(N_WORKERS, n_real)
        a = jnp.pad(a, ((0, 0), (0, E_PER_WORKER - n_real)))
        return a.reshape(N_WORKERS, N_CHUNKS, CHUNK)

    return jnp.stack([shape(src), shape(dst)], axis=2), shape(w)


def _sc_aggregate(h, packed, packed_w):
    mesh = plsc.VectorSubcoreMesh(core_axis_name="c", subcore_axis_name="s")

    @functools.partial(
        pl.kernel,
        mesh=mesh,
        out_type=jax.ShapeDtypeStruct((N_CORES, N_NODES, NFEAT), jnp.float32),
        scratch_types=[
            pltpu.VMEM_SHARED((N_NODES, NFEAT), jnp.float32),
            pltpu.VMEM((S_PHASE, 2, CHUNK), jnp.int32),
            pltpu.VMEM((S_PHASE, CHUNK), jnp.float32),
            pltpu.VMEM((CHUNK, NFEAT), jnp.float32),
            pltpu.VMEM((CHUNK, NFEAT), jnp.float32),
            pltpu.VMEM((CHUNK, NFEAT), jnp.float32),
            pltpu.SemaphoreType.DMA,
            pltpu.SemaphoreType.DMA,
            pltpu.SemaphoreType.DMA,
            pltpu.SemaphoreType.DMA,
            pltpu.SemaphoreType.DMA,
            pltpu.SemaphoreType.DMA,
        ],
        compiler_params=pltpu.CompilerParams(use_tc_tiling_on_sc=False),
    )
    def agg(h_hbm, packed_hbm, packedw_hbm, out_hbm,
            acc, stage, stage_w, m0, m1, m2, g0, g1, g2, s0, s1, s2):
        c = lax.axis_index("c")
        s = lax.axis_index("s")
        wid = c * N_SUBCORES + s
        msgs = (m0, m1, m2)
        gsem = (g0, g1, g2)
        ssem = (s0, s1, s2)
        zero16 = jnp.zeros((LANES,), jnp.float32)

        def zero_row(r, carry):
            for f in range(F_CHUNKS):
                m0[r, pl.ds(f * LANES, LANES)] = zero16
            return carry

        lax.fori_loop(0, CHUNK, zero_row, 0)
        row0 = s * ROWS_PER_TILE
        for jz in range(ROWS_PER_TILE // CHUNK):
            pltpu.sync_copy(m0, acc.at[pl.ds(row0 + jz * CHUNK, CHUNK)])
        rem0 = (ROWS_PER_TILE // CHUNK) * CHUNK
        rem = ROWS_PER_TILE - rem0
        if rem:
            pltpu.sync_copy(m0.at[pl.ds(0, rem)],
                            acc.at[pl.ds(row0 + rem0, rem)])

        @pl.when(s == N_SUBCORES - 1)
        def _zero_tail():
            pltpu.sync_copy(m0.at[pl.ds(0, TAIL_ROWS)],
                            acc.at[pl.ds(TAIL_ROW0, TAIL_ROWS)])

        plsc.subcore_barrier()

        def issue_gather(j, b):
            pltpu.async_copy(h_hbm.at[stage.at[j, 0]], msgs[b], gsem[b])

        def wait_gather(j, b):
            pltpu.make_async_copy(h_hbm.at[stage.at[j, 0]],
                                  msgs[b], gsem[b]).wait()

        def issue_scatter(j, b):
            pltpu.async_copy(msgs[b], acc.at[stage.at[j, 1]], ssem[b],
                             add=True)

        def wait_scatter(j, b):
            pltpu.make_async_copy(msgs[b], acc.at[stage.at[j, 1]],
                                  ssem[b]).wait()

        def scale(j, b):
            mb = msgs[b]

            @plsc.parallel_loop(0, G_CHUNKS, unroll=2)
            def scale_g(g):
                wv = stage_w[j, pl.ds(g * LANES, LANES)]
                for r2 in range(LANES):
                    e = g * LANES + r2
                    wr = wv[r2]
                    for f in range(F_CHUNKS):
                        sl = pl.ds(f * LANES, LANES)
                        mb[e, sl] = mb[e, sl] * wr

        def process(j, b):
            wait_gather(j, b)
            scale(j, b)
            issue_scatter(j, b)

        def phase_body(ph, carry):
            pltpu.sync_copy(packed_hbm.at[wid, pl.ds(ph * S_PHASE, S_PHASE)],
                            stage)
            pltpu.sync_copy(packedw_hbm.at[wid, pl.ds(ph * S_PHASE, S_PHASE)],
                            stage_w)
            issue_gather(0, 0)
            issue_gather(1, 1)
            process(0, 0)
            issue_gather(2, 2)
            process(1, 1)
            wait_scatter(0, 0)
            issue_gather(3, 0)
            process(2, 2)
            wait_scatter(1, 1)
            issue_gather(4, 1)

            def triple(t, cc):
                for r in range(3):
                    j = 3 + 3 * t + r
                    b = r
                    process(j, b)
                    nb = (r + 2) % 3
                    wait_scatter(j - 1, nb)
                    issue_gather(j + 2, nb)
                return cc

            lax.fori_loop(0, (S_PHASE - 5) // 3, triple, 0)

            process(S_PHASE - 2, 0)
            process(S_PHASE - 1, 1)
            wait_scatter(S_PHASE - 3, 2)
            wait_scatter(S_PHASE - 2, 0)
            wait_scatter(S_PHASE - 1, 1)
            return carry

        lax.fori_loop(0, N_PHASES, phase_body, 0)

        plsc.subcore_barrier()
        pltpu.sync_copy(acc.at[pl.ds(row0, ROWS_PER_TILE)],
                        out_hbm.at[c, pl.ds(row0, ROWS_PER_TILE)])

        @pl.when(s == N_SUBCORES - 1)
        def _copy_tail():
            pltpu.sync_copy(acc.at[pl.ds(TAIL_ROW0, TAIL_ROWS)],
                            out_hbm.at[c, pl.ds(TAIL_ROW0, TAIL_ROWS)])

    return agg(h, packed, packed_w)


def _combine_matmul(parts, W, relu):
    rows_blk = 1000

    def mm(p_ref, w_ref, o_ref):
        a = p_ref[0] + p_ref[1]
        y = jnp.dot(a, w_ref[...], preferred_element_type=jnp.float32)
        if relu:
            y = jnp.maximum(y, 0.0)
        o_ref[...] = y

    return pl.pallas_call(
        mm,
        grid=(N_NODES // rows_blk,),
        in_specs=[
            pl.BlockSpec((N_CORES, rows_blk, NFEAT), lambda i: (0, i, 0)),
            pl.BlockSpec((NFEAT, NFEAT), lambda i: (0, 0)),
        ],
        out_specs=pl.BlockSpec((rows_blk, NFEAT), lambda i: (i, 0)),
        out_shape=jax.ShapeDtypeStruct((N_NODES, NFEAT), jnp.float32),
    )(parts, W)


def kernel(x, edge_index1, edge_index2, edge_weight1, edge_weight2, W1, W2):
    packed1, pw1 = _pack_edges(edge_index1[0].astype(jnp.int32),
                               edge_index1[1].astype(jnp.int32), edge_weight1)
    packed2, pw2 = _pack_edges(edge_index2[0].astype(jnp.int32),
                               edge_index2[1].astype(jnp.int32), edge_weight2)

    p1 = _sc_aggregate(x, packed1, pw1)
    h = _combine_matmul(p1, W1, relu=True)
    p2 = _sc_aggregate(h, packed2, pw2)
    return _combine_matmul(p2, W2, relu=False)
